# baseline (speedup 1.0000x reference)
import functools

import jax
import jax.numpy as jnp
from jax import lax
from jax.experimental import pallas as pl
from jax.experimental.pallas import tpu as pltpu
from jax.experimental.pallas import tpu_sc as plsc


TILE = 1024
N_SRC = 4096
N_TAR = 2048
C = 256
B = 4

NW = 32
R_PER_W = (B * N_TAR) // NW
CHUNK = 32
N_CHUNKS = R_PER_W // CHUNK
LANES = 16
FCH = C // LANES


GRP = 128
N_GRP = N_SRC // GRP
KEY_MASK = -32
INT_INF = 0x7FFFFFFF


def _knn_tile(batch0, loc_tar_ref, loc_src_t_ref, idx_ref, w_ref):
    b = pl.program_id(0)
    lt = loc_tar_ref[0]
    lst = loc_src_t_ref[0]

    tx = lt[:, 0:1]
    ty = lt[:, 1:2]

    s1 = jnp.full((TILE, GRP), INT_INF, dtype=jnp.int32)
    s2 = s1
    s3 = s1

    for g in range(N_GRP):
        sx = lst[0:1, g * GRP:(g + 1) * GRP]
        sy = lst[1:2, g * GRP:(g + 1) * GRP]
        dx = tx - sx
        dy = ty - sy
        d = dx * dx + dy * dy
        key = (jax.lax.bitcast_convert_type(d, jnp.int32) & KEY_MASK) | g
        lt1 = key < s1
        lt2 = key < s2
        lt3 = key < s3
        s3 = jnp.where(lt2, s2, jnp.where(lt3, key, s3))
        s2 = jnp.where(lt1, s1, jnp.where(lt2, key, s2))
        s1 = jnp.where(lt1, key, s1)

    cand = jnp.concatenate([s1, s2, s3], axis=1)
    lane = jax.lax.broadcasted_iota(jnp.int32, (TILE, GRP), 1)
    lane3 = jnp.concatenate([lane, lane, lane], axis=1)
    cand_idx = ((cand & 31) << 7) | lane3

    d_list = []
    idx_list = []
    for _ in range(3):
        m = jnp.min(cand, axis=1)
        ci = jnp.min(jnp.where(cand == m[:, None], cand_idx, INT_INF), axis=1)
        cand = jnp.where(cand_idx == ci[:, None], INT_INF, cand)
        d_list.append(jax.lax.bitcast_convert_type(m & KEY_MASK, jnp.float32))
        idx_list.append(ci)

    d3 = jnp.stack(d_list, axis=1)
    dist_recip = 1.0 / (d3 + 1e-06)
    one_mask = d3 == 0.0
    zero_mask = jnp.sum(one_mask, axis=-1) > 0
    dist_recip = jnp.where(zero_mask[:, None], 0.0, dist_recip)
    dist_recip = jnp.where(one_mask, 1.0, dist_recip)
    norm = jnp.sum(dist_recip, axis=1, keepdims=True)
    weight = dist_recip / norm

    idx3 = jnp.stack(idx_list, axis=1) + (b + batch0) * N_SRC
    idx_ref[0] = idx3.astype(jnp.int32)
    w_ref[0] = jnp.concatenate(
        [jnp.broadcast_to(weight[:, k:k + 1], (TILE, LANES)) for k in range(3)],
        axis=1,
    )


def _tc_knn(loc_src_t, loc_tar, batch0):
    nb = loc_tar.shape[0]
    grid = (nb, N_TAR // TILE)
    return pl.pallas_call(
        functools.partial(_knn_tile, batch0),
        grid=grid,
        in_specs=[
            pl.BlockSpec((1, TILE, 2), lambda b, t: (b, t, 0)),
            pl.BlockSpec((1, 2, N_SRC), lambda b, t: (b, 0, 0)),
        ],
        out_specs=[
            pl.BlockSpec((1, TILE, 3), lambda b, t: (b, t, 0)),
            pl.BlockSpec((1, TILE, 3 * LANES), lambda b, t: (b, t, 0)),
        ],
        out_shape=[
            jax.ShapeDtypeStruct((nb, N_TAR, 3), jnp.int32),
            jax.ShapeDtypeStruct((nb, N_TAR, 3 * LANES), jnp.float32),
        ],
    )(loc_tar, loc_src_t)


def _sc_body(r_per_w, n_chunks,
             idx_hbm, w_hbm, table_hbm, out_hbm,
             idx_all, w_all, rows0, rows1, out0, out1,
             gsem0, gsem1, ssem0, ssem1):
    cid = lax.axis_index("c")
    sid = lax.axis_index("s")
    wid = sid * 2 + cid
    base = wid * r_per_w

    rows = (rows0, rows1)
    outs = (out0, out1)
    gsems = (gsem0, gsem1)
    ssems = (ssem0, ssem1)

    pltpu.sync_copy(idx_hbm.at[wid], idx_all)
    pltpu.sync_copy(w_hbm.at[wid], w_all)

    def compute_chunk(j, rows_v, out_v):
        def row_body(r, _):
            wr = j * CHUNK + r
            w0 = w_all[wr, pl.ds(0, LANES)]
            w1 = w_all[wr, pl.ds(LANES, LANES)]
            w2 = w_all[wr, pl.ds(2 * LANES, LANES)]
            r3 = r * 3
            for f in range(FCH):
                sl = pl.ds(f * LANES, LANES)
                out_v[r, sl] = (w0 * rows_v[r3, sl]
                                + w1 * rows_v[r3 + 1, sl]
                                + w2 * rows_v[r3 + 2, sl])
            return 0

        lax.fori_loop(0, CHUNK, row_body, 0)

    gh = [None, None]
    sh = [None, None]
    gh[0] = pltpu.async_copy(table_hbm.at[idx_all.at[0]], rows[0], gsems[0])
    for j in range(n_chunks):
        cur = j % 2
        nxt = 1 - cur
        if j + 1 < n_chunks:
            gh[nxt] = pltpu.async_copy(
                table_hbm.at[idx_all.at[j + 1]], rows[nxt], gsems[nxt])
        gh[cur].wait()
        if j >= 2:
            sh[cur].wait()
        compute_chunk(j, rows[cur], outs[cur])
        sh[cur] = pltpu.async_copy(
            outs[cur], out_hbm.at[pl.ds(base + j * CHUNK, CHUNK)], ssems[cur])
    if n_chunks >= 2:
        sh[(n_chunks - 2) % 2].wait()
    sh[(n_chunks - 1) % 2].wait()


def _sc_gather(idx_c, w_c, table, n_rows):
    r_per_w = n_rows // NW
    n_chunks = r_per_w // CHUNK
    mesh = plsc.VectorSubcoreMesh(core_axis_name="c", subcore_axis_name="s")
    fn = functools.partial(
        pl.kernel,
        mesh=mesh,
        out_type=jax.ShapeDtypeStruct((n_rows, C), jnp.float32),
        scratch_types=[
            pltpu.VMEM((n_chunks, CHUNK * 3), jnp.int32),
            pltpu.VMEM((r_per_w, 3 * LANES), jnp.float32),
            pltpu.VMEM((CHUNK * 3, C), jnp.float32),
            pltpu.VMEM((CHUNK * 3, C), jnp.float32),
            pltpu.VMEM((CHUNK, C), jnp.float32),
            pltpu.VMEM((CHUNK, C), jnp.float32),
            pltpu.SemaphoreType.DMA,
            pltpu.SemaphoreType.DMA,
            pltpu.SemaphoreType.DMA,
            pltpu.SemaphoreType.DMA,
        ],
    )(functools.partial(_sc_body, r_per_w, n_chunks))
    return fn(idx_c, w_c, table)


def _half(loc_src_t, loc_tar_h, table, batch0, nb):
    n_rows = nb * N_TAR
    r_per_w = n_rows // NW
    n_chunks = r_per_w // CHUNK
    idx3, w48 = _tc_knn(loc_src_t, loc_tar_h, batch0)
    idx_c = idx3.reshape(NW, n_chunks, CHUNK * 3)
    w_c = w48.reshape(NW, r_per_w, 3 * LANES)
    return _sc_gather(idx_c, w_c, table, n_rows)


@jax.jit
def kernel(x_src, loc_src, loc_tar):
    table = x_src.reshape(B * N_SRC, C)
    loc_src_t = jnp.transpose(loc_src, (0, 2, 1))
    outs = [_half(loc_src_t[b:b + 1], loc_tar[b:b + 1], table, b, 1)
            for b in range(B)]
    out = jnp.concatenate(outs, axis=0)
    return out.reshape(B, N_TAR, C)


# speedup vs baseline: 1.0892x; 1.0892x over previous
import functools

import jax
import jax.numpy as jnp
from jax import lax
from jax.experimental import pallas as pl
from jax.experimental.pallas import tpu as pltpu
from jax.experimental.pallas import tpu_sc as plsc


TILE = 1024
N_SRC = 4096
N_TAR = 2048
C = 256
B = 4

NW = 32
R_PER_W = (B * N_TAR) // NW
CHUNK = 32
N_CHUNKS = R_PER_W // CHUNK
LANES = 16
FCH = C // LANES


GRP = 128
N_GRP = N_SRC // GRP
KEY_MASK = -32
INT_INF = 0x7FFFFFFF


def _knn_tile(batch0, loc_tar_ref, loc_src_t_ref, idx_ref, w_ref):
    b = pl.program_id(0)
    lt = loc_tar_ref[0]
    lst = loc_src_t_ref[0]

    tx = lt[:, 0:1]
    ty = lt[:, 1:2]

    s1 = jnp.full((TILE, GRP), INT_INF, dtype=jnp.int32)
    s2 = s1
    s3 = s1

    for g in range(N_GRP):
        sx = lst[0:1, g * GRP:(g + 1) * GRP]
        sy = lst[1:2, g * GRP:(g + 1) * GRP]
        dx = tx - sx
        dy = ty - sy
        d = dx * dx + dy * dy
        key = (jax.lax.bitcast_convert_type(d, jnp.int32) & KEY_MASK) | g
        lt1 = key < s1
        lt2 = key < s2
        lt3 = key < s3
        s3 = jnp.where(lt2, s2, jnp.where(lt3, key, s3))
        s2 = jnp.where(lt1, s1, jnp.where(lt2, key, s2))
        s1 = jnp.where(lt1, key, s1)

    cand = jnp.concatenate([s1, s2, s3], axis=1)
    lane = jax.lax.broadcasted_iota(jnp.int32, (TILE, GRP), 1)
    lane3 = jnp.concatenate([lane, lane, lane], axis=1)
    cand_idx = ((cand & 31) << 7) | lane3

    d_list = []
    idx_list = []
    for _ in range(3):
        m = jnp.min(cand, axis=1)
        ci = jnp.min(jnp.where(cand == m[:, None], cand_idx, INT_INF), axis=1)
        cand = jnp.where(cand_idx == ci[:, None], INT_INF, cand)
        d_list.append(jax.lax.bitcast_convert_type(m & KEY_MASK, jnp.float32))
        idx_list.append(ci)

    d3 = jnp.stack(d_list, axis=1)
    dist_recip = 1.0 / (d3 + 1e-06)
    one_mask = d3 == 0.0
    zero_mask = jnp.sum(one_mask, axis=-1) > 0
    dist_recip = jnp.where(zero_mask[:, None], 0.0, dist_recip)
    dist_recip = jnp.where(one_mask, 1.0, dist_recip)
    norm = jnp.sum(dist_recip, axis=1, keepdims=True)
    weight = dist_recip / norm

    idx3 = jnp.stack(idx_list, axis=1) + (b + batch0) * N_SRC
    idx_ref[0] = idx3.astype(jnp.int32)
    w_ref[0] = jnp.concatenate(
        [jnp.broadcast_to(weight[:, k:k + 1], (TILE, LANES)) for k in range(3)],
        axis=1,
    )


def _tc_knn(loc_src_t, loc_tar, batch0):
    nb = loc_tar.shape[0]
    grid = (nb, N_TAR // TILE)
    return pl.pallas_call(
        functools.partial(_knn_tile, batch0),
        grid=grid,
        in_specs=[
            pl.BlockSpec((1, TILE, 2), lambda b, t: (b, t, 0)),
            pl.BlockSpec((1, 2, N_SRC), lambda b, t: (b, 0, 0)),
        ],
        out_specs=[
            pl.BlockSpec((1, TILE, 3), lambda b, t: (b, t, 0)),
            pl.BlockSpec((1, TILE, 3 * LANES), lambda b, t: (b, t, 0)),
        ],
        out_shape=[
            jax.ShapeDtypeStruct((nb, N_TAR, 3), jnp.int32),
            jax.ShapeDtypeStruct((nb, N_TAR, 3 * LANES), jnp.float32),
        ],
    )(loc_tar, loc_src_t)


def _sc_body(r_per_w, n_chunks,
             idx_hbm, w_hbm, table_hbm, out_hbm,
             idx_all, w_all, rows0, rows1, out0, out1,
             gsem0, gsem1, ssem0, ssem1):
    cid = lax.axis_index("c")
    sid = lax.axis_index("s")
    wid = sid * 2 + cid
    base = wid * r_per_w

    rows = (rows0, rows1)
    outs = (out0, out1)
    gsems = (gsem0, gsem1)
    ssems = (ssem0, ssem1)

    pltpu.sync_copy(idx_hbm.at[wid], idx_all)
    pltpu.sync_copy(w_hbm.at[wid], w_all)

    def compute_chunk(j, rows_v, out_v):
        @plsc.parallel_loop(0, CHUNK, unroll=4)
        def row_body(r):
            wr = j * CHUNK + r
            w0 = w_all[wr, pl.ds(0, LANES)]
            w1 = w_all[wr, pl.ds(LANES, LANES)]
            w2 = w_all[wr, pl.ds(2 * LANES, LANES)]
            r3 = r * 3
            for f in range(FCH):
                sl = pl.ds(f * LANES, LANES)
                out_v[r, sl] = (w0 * rows_v[r3, sl]
                                + w1 * rows_v[r3 + 1, sl]
                                + w2 * rows_v[r3 + 2, sl])

    gh = [None, None]
    sh = [None, None]
    gh[0] = pltpu.async_copy(table_hbm.at[idx_all.at[0]], rows[0], gsems[0])
    for j in range(n_chunks):
        cur = j % 2
        nxt = 1 - cur
        if j + 1 < n_chunks:
            gh[nxt] = pltpu.async_copy(
                table_hbm.at[idx_all.at[j + 1]], rows[nxt], gsems[nxt])
        gh[cur].wait()
        if j >= 2:
            sh[cur].wait()
        compute_chunk(j, rows[cur], outs[cur])
        sh[cur] = pltpu.async_copy(
            outs[cur], out_hbm.at[pl.ds(base + j * CHUNK, CHUNK)], ssems[cur])
    if n_chunks >= 2:
        sh[(n_chunks - 2) % 2].wait()
    sh[(n_chunks - 1) % 2].wait()


def _sc_gather(idx_c, w_c, table, n_rows):
    r_per_w = n_rows // NW
    n_chunks = r_per_w // CHUNK
    mesh = plsc.VectorSubcoreMesh(core_axis_name="c", subcore_axis_name="s")
    fn = functools.partial(
        pl.kernel,
        mesh=mesh,
        out_type=jax.ShapeDtypeStruct((n_rows, C), jnp.float32),
        scratch_types=[
            pltpu.VMEM((n_chunks, CHUNK * 3), jnp.int32),
            pltpu.VMEM((r_per_w, 3 * LANES), jnp.float32),
            pltpu.VMEM((CHUNK * 3, C), jnp.float32),
            pltpu.VMEM((CHUNK * 3, C), jnp.float32),
            pltpu.VMEM((CHUNK, C), jnp.float32),
            pltpu.VMEM((CHUNK, C), jnp.float32),
            pltpu.SemaphoreType.DMA,
            pltpu.SemaphoreType.DMA,
            pltpu.SemaphoreType.DMA,
            pltpu.SemaphoreType.DMA,
        ],
    )(functools.partial(_sc_body, r_per_w, n_chunks))
    return fn(idx_c, w_c, table)


def _half(loc_src_t, loc_tar_h, table, batch0, nb):
    n_rows = nb * N_TAR
    r_per_w = n_rows // NW
    n_chunks = r_per_w // CHUNK
    idx3, w48 = _tc_knn(loc_src_t, loc_tar_h, batch0)
    idx_c = idx3.reshape(NW, n_chunks, CHUNK * 3)
    w_c = w48.reshape(NW, r_per_w, 3 * LANES)
    return _sc_gather(idx_c, w_c, table, n_rows)


@jax.jit
def kernel(x_src, loc_src, loc_tar):
    table = x_src.reshape(B * N_SRC, C)
    loc_src_t = jnp.transpose(loc_src, (0, 2, 1))
    hb = B // 2
    out_a = _half(loc_src_t[:hb], loc_tar[:hb], table, 0, hb)
    out_b = _half(loc_src_t[hb:], loc_tar[hb:], table, hb, hb)
    out = jnp.concatenate([out_a, out_b], axis=0)
    return out.reshape(B, N_TAR, C)


# speedup vs baseline: 1.1254x; 1.0332x over previous
import functools

import jax
import jax.numpy as jnp
from jax import lax
from jax.experimental import pallas as pl
from jax.experimental.pallas import tpu as pltpu
from jax.experimental.pallas import tpu_sc as plsc


TILE = 1024
N_SRC = 4096
N_TAR = 2048
C = 256
B = 4

NW = 32
R_PER_W = (B * N_TAR) // NW
CHUNK = 32
N_CHUNKS = R_PER_W // CHUNK
LANES = 16
FCH = C // LANES


GRP = 128
N_GRP = N_SRC // GRP
KEY_MASK = -32
INT_INF = 0x7FFFFFFF


def _knn_tile(batch0, loc_tar_ref, loc_src_t_ref, idx_ref, w_ref):
    b = pl.program_id(0)
    lt = loc_tar_ref[0]
    lst = loc_src_t_ref[0]

    tx = lt[:, 0:1]
    ty = lt[:, 1:2]

    s1 = jnp.full((TILE, GRP), INT_INF, dtype=jnp.int32)
    s2 = s1
    s3 = s1

    for g in range(N_GRP):
        sx = lst[0:1, g * GRP:(g + 1) * GRP]
        sy = lst[1:2, g * GRP:(g + 1) * GRP]
        dx = tx - sx
        dy = ty - sy
        d = dx * dx + dy * dy
        key = (jax.lax.bitcast_convert_type(d, jnp.int32) & KEY_MASK) | g
        lt1 = key < s1
        lt2 = key < s2
        lt3 = key < s3
        s3 = jnp.where(lt2, s2, jnp.where(lt3, key, s3))
        s2 = jnp.where(lt1, s1, jnp.where(lt2, key, s2))
        s1 = jnp.where(lt1, key, s1)

    lane = jax.lax.broadcasted_iota(jnp.int32, (TILE, GRP), 1)
    w_cur = s1
    n1 = s2
    n2 = s3
    d_list = []
    idx_list = []
    for _ in range(3):
        m = jnp.min(w_cur, axis=1)
        widx = ((w_cur & 31) << 7) | lane
        ci = jnp.min(jnp.where(w_cur == m[:, None], widx, INT_INF), axis=1)
        hit = lane == (ci & (GRP - 1))[:, None]
        w_cur = jnp.where(hit, n1, w_cur)
        n1 = jnp.where(hit, n2, n1)
        n2 = jnp.where(hit, INT_INF, n2)
        d_list.append(jax.lax.bitcast_convert_type(m & KEY_MASK, jnp.float32))
        idx_list.append(ci)

    d3 = jnp.stack(d_list, axis=1)
    dist_recip = 1.0 / (d3 + 1e-06)
    one_mask = d3 == 0.0
    zero_mask = jnp.sum(one_mask, axis=-1) > 0
    dist_recip = jnp.where(zero_mask[:, None], 0.0, dist_recip)
    dist_recip = jnp.where(one_mask, 1.0, dist_recip)
    norm = jnp.sum(dist_recip, axis=1, keepdims=True)
    weight = dist_recip / norm

    idx3 = jnp.stack(idx_list, axis=1) + (b + batch0) * N_SRC
    idx_ref[0] = idx3.astype(jnp.int32)
    w_ref[0] = jnp.concatenate(
        [jnp.broadcast_to(weight[:, k:k + 1], (TILE, LANES)) for k in range(3)],
        axis=1,
    )


def _tc_knn(loc_src_t, loc_tar, batch0):
    nb = loc_tar.shape[0]
    grid = (nb, N_TAR // TILE)
    return pl.pallas_call(
        functools.partial(_knn_tile, batch0),
        grid=grid,
        in_specs=[
            pl.BlockSpec((1, TILE, 2), lambda b, t: (b, t, 0)),
            pl.BlockSpec((1, 2, N_SRC), lambda b, t: (b, 0, 0)),
        ],
        out_specs=[
            pl.BlockSpec((1, TILE, 3), lambda b, t: (b, t, 0)),
            pl.BlockSpec((1, TILE, 3 * LANES), lambda b, t: (b, t, 0)),
        ],
        out_shape=[
            jax.ShapeDtypeStruct((nb, N_TAR, 3), jnp.int32),
            jax.ShapeDtypeStruct((nb, N_TAR, 3 * LANES), jnp.float32),
        ],
    )(loc_tar, loc_src_t)


def _sc_body(r_per_w, n_chunks,
             idx_hbm, w_hbm, table_hbm, out_hbm,
             idx_all, w_all, rows0, rows1, out0, out1,
             gsem0, gsem1, ssem0, ssem1):
    cid = lax.axis_index("c")
    sid = lax.axis_index("s")
    wid = sid * 2 + cid
    base = wid * r_per_w

    rows = (rows0, rows1)
    outs = (out0, out1)
    gsems = (gsem0, gsem1)
    ssems = (ssem0, ssem1)

    pltpu.sync_copy(idx_hbm.at[wid], idx_all)
    pltpu.sync_copy(w_hbm.at[wid], w_all)

    def compute_chunk(j, rows_v, out_v):
        @plsc.parallel_loop(0, CHUNK, unroll=4)
        def row_body(r):
            wr = j * CHUNK + r
            w0 = w_all[wr, pl.ds(0, LANES)]
            w1 = w_all[wr, pl.ds(LANES, LANES)]
            w2 = w_all[wr, pl.ds(2 * LANES, LANES)]
            r3 = r * 3
            for f in range(FCH):
                sl = pl.ds(f * LANES, LANES)
                out_v[r, sl] = (w0 * rows_v[r3, sl]
                                + w1 * rows_v[r3 + 1, sl]
                                + w2 * rows_v[r3 + 2, sl])

    gh = [None, None]
    sh = [None, None]
    gh[0] = pltpu.async_copy(table_hbm.at[idx_all.at[0]], rows[0], gsems[0])
    for j in range(n_chunks):
        cur = j % 2
        nxt = 1 - cur
        if j + 1 < n_chunks:
            gh[nxt] = pltpu.async_copy(
                table_hbm.at[idx_all.at[j + 1]], rows[nxt], gsems[nxt])
        gh[cur].wait()
        if j >= 2:
            sh[cur].wait()
        compute_chunk(j, rows[cur], outs[cur])
        sh[cur] = pltpu.async_copy(
            outs[cur], out_hbm.at[pl.ds(base + j * CHUNK, CHUNK)], ssems[cur])
    if n_chunks >= 2:
        sh[(n_chunks - 2) % 2].wait()
    sh[(n_chunks - 1) % 2].wait()


def _sc_gather(idx_c, w_c, table, n_rows):
    r_per_w = n_rows // NW
    n_chunks = r_per_w // CHUNK
    mesh = plsc.VectorSubcoreMesh(core_axis_name="c", subcore_axis_name="s")
    fn = functools.partial(
        pl.kernel,
        mesh=mesh,
        out_type=jax.ShapeDtypeStruct((n_rows, C), jnp.float32),
        scratch_types=[
            pltpu.VMEM((n_chunks, CHUNK * 3), jnp.int32),
            pltpu.VMEM((r_per_w, 3 * LANES), jnp.float32),
            pltpu.VMEM((CHUNK * 3, C), jnp.float32),
            pltpu.VMEM((CHUNK * 3, C), jnp.float32),
            pltpu.VMEM((CHUNK, C), jnp.float32),
            pltpu.VMEM((CHUNK, C), jnp.float32),
            pltpu.SemaphoreType.DMA,
            pltpu.SemaphoreType.DMA,
            pltpu.SemaphoreType.DMA,
            pltpu.SemaphoreType.DMA,
        ],
    )(functools.partial(_sc_body, r_per_w, n_chunks))
    return fn(idx_c, w_c, table)


def _half(loc_src_t, loc_tar_h, table, batch0, nb):
    n_rows = nb * N_TAR
    r_per_w = n_rows // NW
    n_chunks = r_per_w // CHUNK
    idx3, w48 = _tc_knn(loc_src_t, loc_tar_h, batch0)
    idx_c = idx3.reshape(NW, n_chunks, CHUNK * 3)
    w_c = w48.reshape(NW, r_per_w, 3 * LANES)
    return _sc_gather(idx_c, w_c, table, n_rows)


@jax.jit
def kernel(x_src, loc_src, loc_tar):
    table = x_src.reshape(B * N_SRC, C)
    loc_src_t = jnp.transpose(loc_src, (0, 2, 1))
    hb = B // 2
    out_a = _half(loc_src_t[:hb], loc_tar[:hb], table, 0, hb)
    out_b = _half(loc_src_t[hb:], loc_tar[hb:], table, hb, hb)
    out = jnp.concatenate([out_a, out_b], axis=0)
    return out.reshape(B, N_TAR, C)
